# baseline (speedup 1.0000x reference)
import functools

import jax
import jax.numpy as jnp
from jax import lax
from jax.experimental import pallas as pl
from jax.experimental.pallas import tpu as pltpu
from jax.experimental.pallas import tpu_sc as plsc

BATCH = 4096
HIST = 200
DIM = 64
N = BATCH * HIST

NC = 2
NS = 16
NW = NC * NS
PER_W = N // NW

CHUNK = 400
NCHUNK = PER_W // CHUNK

assert NCHUNK * CHUNK == PER_W

_MESH = plsc.VectorSubcoreMesh(core_axis_name="c", subcore_axis_name="s")


@functools.partial(
    pl.kernel,
    mesh=_MESH,
    out_type=jax.ShapeDtypeStruct((N, DIM), jnp.float32),
    compiler_params=pltpu.CompilerParams(use_tc_tiling_on_sc=False),
    scratch_types=[
        pltpu.VMEM((CHUNK,), jnp.int32),
        pltpu.VMEM((CHUNK, DIM), jnp.float32),
        pltpu.SemaphoreType.DMA,
    ],
)
def _sc_gather(idx_hbm, table_hbm, out_hbm, idx_v, rows_v, sem):
    wid = lax.axis_index("s") * NC + lax.axis_index("c")
    base = wid * NCHUNK

    def body(c, carry):
        pltpu.sync_copy(idx_hbm.at[base + c], idx_v)
        pltpu.async_copy(table_hbm.at[idx_v], rows_v, sem)
        pltpu.make_async_copy(
            table_hbm.at[pl.ds(0, CHUNK)], rows_v, sem
        ).wait()
        pltpu.sync_copy(
            rows_v, out_hbm.at[pl.ds((base + c) * CHUNK, CHUNK)]
        )
        return carry

    lax.fori_loop(0, NCHUNK, body, 0)


def kernel(x, table):
    idx = x.reshape(N // CHUNK, CHUNK).astype(jnp.int32)
    return _sc_gather(idx, table).reshape(BATCH, HIST, DIM)


# speedup vs baseline: 1.0582x; 1.0582x over previous
import functools

import jax
import jax.numpy as jnp
from jax import lax
from jax.experimental import pallas as pl
from jax.experimental.pallas import tpu as pltpu
from jax.experimental.pallas import tpu_sc as plsc

BATCH = 4096
HIST = 200
DIM = 64
N = BATCH * HIST

NC = 2
NS = 16
NW = NC * NS
PER_W = N // NW

CHUNK = 800
NCHUNK = PER_W // CHUNK

assert PER_W * NW == N
assert NCHUNK * CHUNK == PER_W
assert NCHUNK % 2 == 0

_MESH = plsc.VectorSubcoreMesh(core_axis_name="c", subcore_axis_name="s")


@functools.partial(
    pl.kernel,
    mesh=_MESH,
    out_type=jax.ShapeDtypeStruct((BATCH, HIST, DIM), jnp.float32),
    compiler_params=pltpu.CompilerParams(use_tc_tiling_on_sc=False),
    scratch_types=[
        pltpu.VMEM((NCHUNK, CHUNK), jnp.int32),
        pltpu.VMEM((2, CHUNK, DIM), jnp.float32),
        pltpu.SemaphoreType.DMA,
        pltpu.SemaphoreType.DMA,
        pltpu.SemaphoreType.DMA,
        pltpu.SemaphoreType.DMA,
    ],
)
def _sc_gather(idx_hbm, table_hbm, out_hbm, idx_v, rows_v, g0, g1, o0, o1):
    wid = lax.axis_index("s") * NC + lax.axis_index("c")
    bpc = CHUNK // HIST
    base_b = wid * (PER_W // HIST)
    gsems = (g0, g1)
    osems = (o0, o1)

    pltpu.sync_copy(idx_hbm.at[pl.ds(wid * NCHUNK, NCHUNK)], idx_v)

    def fire(c, s):
        pltpu.async_copy(table_hbm.at[idx_v.at[c]], rows_v.at[s], gsems[s])

    def drain_gathers(s):
        pltpu.make_async_copy(
            table_hbm.at[pl.ds(0, CHUNK)], rows_v.at[s], gsems[s]
        ).wait()

    class _Store:
        def __init__(self, c, s):
            self.copies = [
                pltpu.make_async_copy(
                    rows_v.at[s, pl.ds(k * HIST, HIST)],
                    out_hbm.at[base_b + c * bpc + k],
                    osems[s],
                )
                for k in range(bpc)
            ]

        def start(self):
            for cp in self.copies:
                cp.start()

        def wait(self):
            for cp in self.copies:
                cp.wait()

    store = _Store

    fire(0, 0)
    fire(1, 1)
    drain_gathers(0)
    store(0, 0).start()

    def body(i, carry):
        c0 = 2 * i + 2
        for b in range(2):
            c = c0 + b
            store(c - 2, b).wait()
            fire(c, b)
            drain_gathers(1 - b)
            store(c - 1, 1 - b).start()
        return carry

    lax.fori_loop(0, (NCHUNK - 2) // 2, body, 0)

    drain_gathers((NCHUNK - 1) % 2)
    store(NCHUNK - 1, (NCHUNK - 1) % 2).start()
    store(NCHUNK - 2, (NCHUNK - 2) % 2).wait()
    store(NCHUNK - 1, (NCHUNK - 1) % 2).wait()


def kernel(x, table):
    idx = x.reshape(N // CHUNK, CHUNK).astype(jnp.int32)
    return _sc_gather(idx, table)
